# Initial kernel scaffold; baseline (speedup 1.0000x reference)
import jax
import jax.numpy as jnp
from jax.experimental import pallas as pl


def kernel(feature2d, depth_mapping_3d, conv_w, bn_gamma, bn_beta):
    raise NotImplementedError("write your pallas kernel here")


# speedup vs baseline: 2.4844x; 2.4844x over previous
import jax
import jax.numpy as jnp
from jax.experimental import pallas as pl
from jax.experimental.pallas import tpu as pltpu

B, C_IN, H, W = 1, 2048, 15, 20
FEAT = 512
SCALE = 16
OH, OW = H * SCALE, W * SCALE
HW = OH * OW
N_VOX = 60 * 36 * 60
VB = 1024


def _stage1_body(w_ref, f2d_ref, gamma_ref, beta_ref, out_ref):
    x = jnp.dot(w_ref[...], f2d_ref[...], preferred_element_type=jnp.float32)
    mean = jnp.mean(x, axis=1, keepdims=True)
    var = jnp.mean(x * x, axis=1, keepdims=True) - mean * mean
    x = (x - mean) * jax.lax.rsqrt(var + 1e-5)
    x = x * gamma_ref[...] + beta_ref[...]
    out_ref[...] = jnp.maximum(x, 0.0)


def _stage2_body(idx_ref, tbl_ref, out_ref):
    v = idx_ref[...]
    valid = v < HW
    vc = jnp.where(valid, v, 0)
    py = vc // OW
    px = vc - py * OW
    fy = py.astype(jnp.float32) * (float(H - 1) / (OH - 1))
    fx = px.astype(jnp.float32) * (float(W - 1) / (OW - 1))
    y0 = jnp.floor(fy)
    x0 = jnp.floor(fx)
    dy = fy - y0
    dx = fx - x0
    y0i = y0.astype(jnp.int32)
    x0i = x0.astype(jnp.int32)
    y1i = jnp.minimum(y0i + 1, H - 1)
    x1i = jnp.minimum(x0i + 1, W - 1)
    ry = jax.lax.broadcasted_iota(jnp.int32, (H, VB), 0)
    rx = jax.lax.broadcasted_iota(jnp.int32, (W, VB), 0)
    wy = (jnp.where(ry == y0i[None, :], 1.0 - dy[None, :], 0.0)
          + jnp.where(ry == y1i[None, :], dy[None, :], 0.0))
    wx = (jnp.where(rx == x0i[None, :], 1.0 - dx[None, :], 0.0)
          + jnp.where(rx == x1i[None, :], dx[None, :], 0.0))
    wy = wy * jnp.where(valid, 1.0, 0.0)[None, :]
    m = (wy[:, None, :] * wx[None, :, :]).reshape(H * W, VB)
    out_ref[...] = jnp.dot(tbl_ref[...], m, preferred_element_type=jnp.float32)


def kernel(feature2d, depth_mapping_3d, conv_w, bn_gamma, bn_beta):
    f2d = feature2d.reshape(C_IN, H * W)
    idx = depth_mapping_3d.reshape(N_VOX).astype(jnp.int32)
    tbl = pl.pallas_call(
        _stage1_body,
        out_shape=jax.ShapeDtypeStruct((FEAT, H * W), jnp.float32),
    )(conv_w, f2d, bn_gamma.reshape(FEAT, 1), bn_beta.reshape(FEAT, 1))

    nblk = pl.cdiv(N_VOX, VB)
    out = pl.pallas_call(
        _stage2_body,
        grid=(nblk,),
        in_specs=[
            pl.BlockSpec((VB,), lambda i: (i,)),
            pl.BlockSpec((FEAT, H * W), lambda i: (0, 0)),
        ],
        out_specs=pl.BlockSpec((FEAT, VB), lambda i: (0, i)),
        out_shape=jax.ShapeDtypeStruct((FEAT, N_VOX), jnp.float32),
    )(idx, tbl)
    return out.reshape(1, FEAT, 60, 36, 60)
